# baseline (speedup 1.0000x reference)
import functools

import jax
import jax.numpy as jnp
from jax.experimental import pallas as pl
from jax.experimental.pallas import tpu as pltpu

_NUM_GATES = 16
_THRESHOLD = 0.8
_CAPACITY_FACTOR = 1.25
_MIN_EXPERT_CAPACITY = 4


def _routing_kernel(x_ref, w_ref, sel_ref, cmod_ref, tri_ref, disp_ref,
                    comb_ref, loss_ref, cnt_ref, gsum_ref, msum_ref, lacc_ref,
                    *, nb, nt, tblk, gsize, cap):
    b = pl.program_id(0)
    t = pl.program_id(1)
    ng = _NUM_GATES

    @pl.when(t == 0)
    def _reset_batch():
        cnt_ref[...] = jnp.zeros_like(cnt_ref)
        gsum_ref[...] = jnp.zeros_like(gsum_ref)
        msum_ref[...] = jnp.zeros_like(msum_ref)

    @pl.when((t == 0) & (b == 0))
    def _reset_all():
        lacc_ref[...] = jnp.zeros_like(lacc_ref)

    xb = x_ref[0]
    w = w_ref[...]
    lt = jax.lax.dot_general(w, xb, (((0,), (1,)), ((), ())),
                             preferred_element_type=jnp.float32)

    m = jnp.max(lt, axis=0, keepdims=True)
    ex = jnp.exp(lt - m)
    pt = ex / jnp.sum(ex, axis=0, keepdims=True)

    e_row = jax.lax.broadcasted_iota(jnp.int32, (ng, tblk), 0)
    prefix = jnp.zeros((ng, tblk), jnp.float32)
    for j in range(ng):
        pj = jnp.broadcast_to(pt[j:j + 1, :], (ng, tblk))
        beats = (pj > pt) | ((pj == pt) & (j < e_row))
        prefix = prefix + jnp.where(beats, pj, 0.0)
    selt = (prefix < _THRESHOLD).astype(jnp.float32)

    sel_sum = jnp.sum(pt * selt, axis=0, keepdims=True)
    wtst = (pt / sel_sum) * selt

    post = cnt_ref[:, 0:1] + jnp.dot(selt, tri_ref[...],
                                     preferred_element_type=jnp.float32)
    cnt_ref[:, 0:1] = cnt_ref[:, 0:1] + jnp.sum(selt, axis=1, keepdims=True)

    capf = float(cap)
    maskt = selt * (post < capf).astype(jnp.float32)
    post = post * maskt
    pos_tok_row = jnp.sum(post, axis=0, keepdims=True)
    mwt = maskt * wtst

    smat = sel_ref[...]
    mw_flat = jax.lax.dot_general(mwt, smat, (((0,), (0,)), ((), ())),
                                  preferred_element_type=jnp.float32)
    ones_col = jnp.full((1, 1), 1.0, jnp.float32)
    pos_tok = jax.lax.dot_general(pos_tok_row, ones_col,
                                  (((0,), (1,)), ((), ())),
                                  preferred_element_type=jnp.float32)

    ohf = cmod_ref[0:1, :] == pos_tok
    comb_ref[0] = jnp.where(ohf, mw_flat, 0.0)
    disp_ref[0] = jnp.where(ohf & (mw_flat > 0.0), 1.0, 0.0)

    gsum_ref[:, 0:1] = gsum_ref[:, 0:1] + jnp.sum(pt, axis=1, keepdims=True)
    msum_ref[:, 0:1] = msum_ref[:, 0:1] + jnp.sum(maskt, axis=1,
                                                  keepdims=True)

    @pl.when(t == nt - 1)
    def _batch_done():
        lacc_ref[...] = lacc_ref[...] + jnp.sum(
            gsum_ref[:, 0:1] * msum_ref[:, 0:1], keepdims=True)

    @pl.when((t == nt - 1) & (b == nb - 1))
    def _finish():
        scale = float(ng) / (float(nb) * float(gsize) * float(gsize))
        loss_ref[...] = lacc_ref[...] * scale


def kernel(x, w_gating):
    b, gsize, dim = x.shape
    ng = _NUM_GATES
    cap = max(min(gsize, int(gsize * _CAPACITY_FACTOR / ng)),
              _MIN_EXPERT_CAPACITY)
    flat = ng * cap
    tblk = 256
    nt = gsize // tblk

    lane = jnp.arange(flat, dtype=jnp.int32)
    smat = (lane[None, :] // cap == jnp.arange(ng, dtype=jnp.int32)[:, None]
            ).astype(jnp.float32)
    cmod = jnp.broadcast_to((lane % cap).astype(jnp.float32), (8, flat))
    tok = jnp.arange(tblk, dtype=jnp.int32)
    tri = (tok[:, None] < tok[None, :]).astype(jnp.float32)

    body = functools.partial(_routing_kernel, nb=b, nt=nt, tblk=tblk,
                             gsize=gsize, cap=cap)
    out_shape = (
        jax.ShapeDtypeStruct((b, gsize, flat), jnp.float32),
        jax.ShapeDtypeStruct((b, gsize, flat), jnp.float32),
        jax.ShapeDtypeStruct((1, 1), jnp.float32),
    )
    grid = (b, nt)
    disp, comb, loss = pl.pallas_call(
        body,
        grid=grid,
        in_specs=[
            pl.BlockSpec((1, tblk, dim), lambda i, j: (i, j, 0)),
            pl.BlockSpec((dim, ng), lambda i, j: (0, 0)),
            pl.BlockSpec((ng, flat), lambda i, j: (0, 0)),
            pl.BlockSpec((8, flat), lambda i, j: (0, 0)),
            pl.BlockSpec((tblk, tblk), lambda i, j: (0, 0)),
        ],
        out_specs=(
            pl.BlockSpec((1, tblk, flat), lambda i, j: (i, j, 0)),
            pl.BlockSpec((1, tblk, flat), lambda i, j: (i, j, 0)),
            pl.BlockSpec((1, 1), lambda i, j: (0, 0)),
        ),
        out_shape=out_shape,
        scratch_shapes=[
            pltpu.VMEM((ng, 128), jnp.float32),
            pltpu.VMEM((ng, 128), jnp.float32),
            pltpu.VMEM((ng, 128), jnp.float32),
            pltpu.VMEM((1, 1), jnp.float32),
        ],
        compiler_params=pltpu.CompilerParams(
            dimension_semantics=("arbitrary", "arbitrary"),
        ),
    )(x, w_gating, smat, cmod, tri)
    return (disp.reshape(b, gsize, ng, cap),
            comb.reshape(b, gsize, ng, cap),
            loss[0, 0])


# speedup vs baseline: 1.0366x; 1.0366x over previous
import functools

import jax
import jax.numpy as jnp
from jax.experimental import pallas as pl
from jax.experimental.pallas import tpu as pltpu

_NUM_GATES = 16
_THRESHOLD = 0.8
_CAPACITY_FACTOR = 1.25
_MIN_EXPERT_CAPACITY = 4


def _routing_kernel(x_ref, w_ref, sel_ref, cmod_ref, tri_ref, disp_ref,
                    comb_ref, loss_ref, cnt_ref, gsum_ref, msum_ref, lacc_ref,
                    *, nb, nt, tblk, gsize, cap):
    b = pl.program_id(0)
    t = pl.program_id(1)
    ng = _NUM_GATES

    @pl.when(t == 0)
    def _reset_batch():
        cnt_ref[...] = jnp.zeros_like(cnt_ref)
        gsum_ref[...] = jnp.zeros_like(gsum_ref)
        msum_ref[...] = jnp.zeros_like(msum_ref)

    @pl.when((t == 0) & (b == 0))
    def _reset_all():
        lacc_ref[...] = jnp.zeros_like(lacc_ref)

    xb = x_ref[0]
    w = w_ref[...]
    lt = jax.lax.dot_general(w, xb, (((0,), (1,)), ((), ())),
                             preferred_element_type=jnp.float32)

    m = jnp.max(lt, axis=0, keepdims=True)
    ex = jnp.exp(lt - m)
    pt = ex / jnp.sum(ex, axis=0, keepdims=True)

    e_row = jax.lax.broadcasted_iota(jnp.int32, (ng, tblk), 0)
    prefix = jnp.zeros((ng, tblk), jnp.float32)
    for j in range(ng):
        pj = jnp.broadcast_to(pt[j:j + 1, :], (ng, tblk))
        beats = (pj > pt) | ((pj == pt) & (j < e_row))
        prefix = prefix + jnp.where(beats, pj, 0.0)
    selt = (prefix < _THRESHOLD).astype(jnp.float32)

    sel_sum = jnp.sum(pt * selt, axis=0, keepdims=True)
    wtst = (pt / sel_sum) * selt

    post = cnt_ref[:, 0:1] + jnp.dot(selt, tri_ref[...],
                                     preferred_element_type=jnp.float32)
    cnt_ref[:, 0:1] = cnt_ref[:, 0:1] + jnp.sum(selt, axis=1, keepdims=True)

    capf = float(cap)
    maskt = selt * (post < capf).astype(jnp.float32)
    post = post * maskt
    pos_tok_row = jnp.sum(post, axis=0, keepdims=True)
    mwt = maskt * wtst

    smat = sel_ref[...]
    mw_flat = jax.lax.dot_general(mwt, smat, (((0,), (0,)), ((), ())),
                                  preferred_element_type=jnp.float32)
    ones_col = jnp.full((1, 1), 1.0, jnp.float32)
    pos_tok = jax.lax.dot_general(pos_tok_row, ones_col,
                                  (((0,), (1,)), ((), ())),
                                  preferred_element_type=jnp.float32)

    ohf = cmod_ref[0:1, :] == pos_tok
    comb_ref[0] = jnp.where(ohf, mw_flat, 0.0)
    disp_ref[0] = jnp.where(ohf & (mw_flat > 0.0), 1.0, 0.0)

    gsum_ref[:, 0:1] = gsum_ref[:, 0:1] + jnp.sum(pt, axis=1, keepdims=True)
    msum_ref[:, 0:1] = msum_ref[:, 0:1] + jnp.sum(maskt, axis=1,
                                                  keepdims=True)

    @pl.when(t == nt - 1)
    def _batch_done():
        lacc_ref[...] = lacc_ref[...] + jnp.sum(
            gsum_ref[:, 0:1] * msum_ref[:, 0:1], keepdims=True)

    @pl.when((t == nt - 1) & (b == nb - 1))
    def _finish():
        scale = float(ng) / (float(nb) * float(gsize) * float(gsize))
        loss_ref[...] = lacc_ref[...] * scale


def kernel(x, w_gating):
    b, gsize, dim = x.shape
    ng = _NUM_GATES
    cap = max(min(gsize, int(gsize * _CAPACITY_FACTOR / ng)),
              _MIN_EXPERT_CAPACITY)
    flat = ng * cap
    tblk = 512
    nt = gsize // tblk

    lane = jnp.arange(flat, dtype=jnp.int32)
    smat = (lane[None, :] // cap == jnp.arange(ng, dtype=jnp.int32)[:, None]
            ).astype(jnp.float32)
    cmod = jnp.broadcast_to((lane % cap).astype(jnp.float32), (8, flat))
    tok = jnp.arange(tblk, dtype=jnp.int32)
    tri = (tok[:, None] < tok[None, :]).astype(jnp.float32)

    body = functools.partial(_routing_kernel, nb=b, nt=nt, tblk=tblk,
                             gsize=gsize, cap=cap)
    out_shape = (
        jax.ShapeDtypeStruct((b, gsize, flat), jnp.float32),
        jax.ShapeDtypeStruct((b, gsize, flat), jnp.float32),
        jax.ShapeDtypeStruct((1, 1), jnp.float32),
    )
    grid = (b, nt)
    disp, comb, loss = pl.pallas_call(
        body,
        grid=grid,
        in_specs=[
            pl.BlockSpec((1, tblk, dim), lambda i, j: (i, j, 0)),
            pl.BlockSpec((dim, ng), lambda i, j: (0, 0)),
            pl.BlockSpec((ng, flat), lambda i, j: (0, 0)),
            pl.BlockSpec((8, flat), lambda i, j: (0, 0)),
            pl.BlockSpec((tblk, tblk), lambda i, j: (0, 0)),
        ],
        out_specs=(
            pl.BlockSpec((1, tblk, flat), lambda i, j: (i, j, 0)),
            pl.BlockSpec((1, tblk, flat), lambda i, j: (i, j, 0)),
            pl.BlockSpec((1, 1), lambda i, j: (0, 0)),
        ),
        out_shape=out_shape,
        scratch_shapes=[
            pltpu.VMEM((ng, 128), jnp.float32),
            pltpu.VMEM((ng, 128), jnp.float32),
            pltpu.VMEM((ng, 128), jnp.float32),
            pltpu.VMEM((1, 1), jnp.float32),
        ],
        compiler_params=pltpu.CompilerParams(
            dimension_semantics=("arbitrary", "arbitrary"),
        ),
    )(x, w_gating, smat, cmod, tri)
    return (disp.reshape(b, gsize, ng, cap),
            comb.reshape(b, gsize, ng, cap),
            loss[0, 0])
